# baseline (speedup 1.0000x reference)
import functools

import jax
import jax.numpy as jnp
from jax import lax
from jax.experimental import pallas as pl
from jax.experimental.pallas import tpu as pltpu
from jax.experimental.pallas import tpu_sc as plsc

N = 10000
E = 320000
NC, NS = 2, 16
NW = NC * NS
NPAD = 10240
C = 80
NCH = 128
NB = 4
NST = 4
EPW = NCH * C
EPAD = NW * EPW
NROW = NPAD // C
RPT = NPAD // NS
BN = 400


def _sc_mesh():
    return plsc.VectorSubcoreMesh(core_axis_name="c", subcore_axis_name="s")


EB = 1280


def _tc_histogram(dst_row, dst_col):
    def body(dr_ref, dc_ref, out_ref):
        i = pl.program_id(0)

        @pl.when(i == 0)
        def _():
            out_ref[...] = jnp.zeros_like(out_ref)

        dr = dr_ref[...]
        dc = dc_ref[...]
        hi_oh = (jax.lax.broadcasted_iota(jnp.int32, (128, EB), 0)
                 == lax.shift_right_logical(dr, 7)).astype(jnp.bfloat16)
        lo_oh = (jax.lax.broadcasted_iota(jnp.int32, (EB, 128), 1)
                 == lax.bitwise_and(dc, 127)).astype(jnp.bfloat16)
        out_ref[...] += jnp.dot(hi_oh, lo_oh,
                                preferred_element_type=jnp.float32)

    return pl.pallas_call(
        body,
        grid=(E // EB,),
        in_specs=[
            pl.BlockSpec((1, EB), lambda i: (0, i)),
            pl.BlockSpec((EB, 1), lambda i: (i, 0)),
        ],
        out_specs=pl.BlockSpec((128, 128), lambda i: (0, 0)),
        out_shape=jax.ShapeDtypeStruct((128, 128), jnp.float32),
    )(dst_row, dst_col)


def _sc_aggregate(u, src3, dst3, zeros_pad):
    @functools.partial(
        pl.kernel,
        out_type=jax.ShapeDtypeStruct((NC, NPAD, 128), jnp.float32),
        mesh=_sc_mesh(),
        scratch_types=(
            [pltpu.VMEM((NCH // NST, C), jnp.int32),
             pltpu.VMEM((NCH // NST, C), jnp.int32)]
            + [pltpu.VMEM((C, 128), jnp.float32) for _ in range(NB)]
            + [pltpu.VMEM_SHARED((NPAD, 128), jnp.float32)]
            + [pltpu.SemaphoreType.DMA for _ in range(2 * NB)]
        ),
    )
    def agg_kernel(u_hbm, src_hbm, dst_hbm, z_hbm, out_hbm,
                   src_v, dst_v, *rest):
        rows = rest[:NB]
        acc = rest[NB]
        sem_g = rest[NB + 1:NB + 1 + NB]
        sem_s = rest[NB + 1 + NB:]
        cid = lax.axis_index("c")
        sid = lax.axis_index("s")
        wid = cid * NS + sid
        nps = NCH // NST
        pltpu.sync_copy(z_hbm.at[pl.ds(sid * RPT, RPT)],
                        acc.at[pl.ds(sid * RPT, RPT)])
        plsc.subcore_barrier()

        def wait_gather(b):
            pltpu.make_async_copy(u_hbm.at[pl.ds(0, C)], rows[b],
                                  sem_g[b]).wait()

        def wait_scatter(b):
            pltpu.make_async_copy(rows[b], acc.at[pl.ds(0, C)],
                                  sem_s[b]).wait()

        def stage(s, carry):
            pltpu.sync_copy(src_hbm.at[wid, pl.ds(s * nps, nps)], src_v)
            pltpu.sync_copy(dst_hbm.at[wid, pl.ds(s * nps, nps)], dst_v)
            for b in range(NB):
                pltpu.async_copy(u_hbm.at[src_v.at[b]], rows[b], sem_g[b])

            def grp(p, c2):
                j = NB * p
                for b in range(NB):
                    wait_gather(b)
                    pltpu.async_copy(rows[b], acc.at[dst_v.at[j + b]],
                                     sem_s[b], add=True)
                for b in range(NB):
                    wait_scatter(b)

                    @pl.when(p + 1 < nps // NB)
                    def _():
                        pltpu.async_copy(u_hbm.at[src_v.at[j + NB + b]],
                                         rows[b], sem_g[b])
                return c2

            lax.fori_loop(0, nps // NB, grp, 0)
            return carry

        lax.fori_loop(0, NST, stage, 0)
        plsc.subcore_barrier()
        pltpu.sync_copy(acc.at[pl.ds(sid * RPT, RPT)],
                        out_hbm.at[cid, pl.ds(sid * RPT, RPT)])

    return agg_kernel(u, src3, dst3, zeros_pad)


def _tc_layer1(degp, x, w1):
    def body(d_ref, x_ref, w_ref, u_ref, dinv_ref):
        deg = d_ref[...] + 1.0
        dinv = lax.rsqrt(jnp.maximum(deg, 1.0))
        xw = jnp.dot(x_ref[...], w_ref[...], preferred_element_type=jnp.float32)
        u_ref[...] = dinv * xw
        dinv_ref[...] = dinv

    return pl.pallas_call(
        body,
        grid=(N // BN,),
        in_specs=[
            pl.BlockSpec((BN, 1), lambda i: (i, 0)),
            pl.BlockSpec((BN, 128), lambda i: (i, 0)),
            pl.BlockSpec((128, 128), lambda i: (0, 0)),
        ],
        out_specs=[
            pl.BlockSpec((BN, 128), lambda i: (i, 0)),
            pl.BlockSpec((BN, 1), lambda i: (i, 0)),
        ],
        out_shape=[
            jax.ShapeDtypeStruct((N, 128), jnp.float32),
            jax.ShapeDtypeStruct((N, 1), jnp.float32),
        ],
    )(degp, x, w1)


def _tc_layer2(sp, u1, dinv, b1, w2):
    def body(s0_ref, s1_ref, u1_ref, dinv_ref, b_ref, w_ref, u2_ref):
        s = s0_ref[0] + s1_ref[0] + u1_ref[...]
        h = jnp.maximum(dinv_ref[...] * s + b_ref[...], 0.0)
        u2 = dinv_ref[...] * jnp.dot(
            h, w_ref[...], preferred_element_type=jnp.float32)
        u2_ref[...] = jnp.concatenate([u2, jnp.zeros_like(u2)], axis=1)

    return pl.pallas_call(
        body,
        grid=(N // BN,),
        in_specs=[
            pl.BlockSpec((1, BN, 128), lambda i: (0, i, 0)),
            pl.BlockSpec((1, BN, 128), lambda i: (1, i, 0)),
            pl.BlockSpec((BN, 128), lambda i: (i, 0)),
            pl.BlockSpec((BN, 1), lambda i: (i, 0)),
            pl.BlockSpec((1, 128), lambda i: (0, 0)),
            pl.BlockSpec((128, 64), lambda i: (0, 0)),
        ],
        out_specs=pl.BlockSpec((BN, 128), lambda i: (i, 0)),
        out_shape=jax.ShapeDtypeStruct((N, 128), jnp.float32),
    )(sp, sp, u1, dinv, b1.reshape(1, 128), w2)


def _tc_layer3(qp, u2, dinv, b2):
    def body(q0_ref, q1_ref, u2_ref, dinv_ref, b_ref, out_ref):
        t = (q0_ref[0] + q1_ref[0] + u2_ref[...])[:, :64]
        z = dinv_ref[...] * t + b_ref[...]
        m = jnp.max(z, axis=1, keepdims=True)
        lse = jnp.log(jnp.sum(jnp.exp(z - m), axis=1, keepdims=True)) + m
        out_ref[...] = z - lse

    return pl.pallas_call(
        body,
        grid=(N // BN,),
        in_specs=[
            pl.BlockSpec((1, BN, 128), lambda i: (0, i, 0)),
            pl.BlockSpec((1, BN, 128), lambda i: (1, i, 0)),
            pl.BlockSpec((BN, 128), lambda i: (i, 0)),
            pl.BlockSpec((BN, 1), lambda i: (i, 0)),
            pl.BlockSpec((1, 64), lambda i: (0, 0)),
        ],
        out_specs=pl.BlockSpec((BN, 64), lambda i: (i, 0)),
        out_shape=jax.ShapeDtypeStruct((N, 64), jnp.float32),
    )(qp, qp, u2, dinv, b2.reshape(1, 64))


def _pad_edges(edge_index):
    ppw = EPW - E // NW
    pad_src = jnp.zeros((NW, ppw), jnp.int32)
    pad_dst = jnp.broadcast_to(N + jnp.arange(ppw, dtype=jnp.int32), (NW, ppw))
    src3 = jnp.concatenate(
        [edge_index[0].reshape(NW, E // NW), pad_src], axis=1).reshape(NW, NCH, C)
    dst3 = jnp.concatenate(
        [edge_index[1].reshape(NW, E // NW), pad_dst], axis=1).reshape(NW, NCH, C)
    return src3, dst3


def kernel(x, edge_index, W1, b1, W2, b2):
    src3, dst3 = _pad_edges(edge_index)
    z128 = jnp.zeros((NPAD, 128), jnp.float32)

    dst = edge_index[1]
    flatdeg = _tc_histogram(dst.reshape(1, E), dst.reshape(E, 1))
    deg = flatdeg.reshape(128 * 128, 1)[:N]
    u1, dinv = _tc_layer1(deg, x, W1)
    sp = _sc_aggregate(u1, src3, dst3, z128)
    u2 = _tc_layer2(sp, u1, dinv, b1, W2)
    qp = _sc_aggregate(u2, src3, dst3, z128)
    return _tc_layer3(qp, u2, dinv, b2)


# speedup vs baseline: 1.1330x; 1.1330x over previous
import functools

import jax
import jax.numpy as jnp
from jax import lax
from jax.experimental import pallas as pl
from jax.experimental.pallas import tpu as pltpu
from jax.experimental.pallas import tpu_sc as plsc

N = 10000
E = 320000
NC, NS = 2, 16
NW = NC * NS
NPAD = 10240
C = 80
NCH = 128
NB = 4
NST = 4
EPW = NCH * C
EPAD = NW * EPW
NROW = NPAD // C
RPT = NPAD // NS
BN = 400


def _sc_mesh():
    return plsc.VectorSubcoreMesh(core_axis_name="c", subcore_axis_name="s")


EB = 32000


def _tc_histogram(dst_row, dst_col):
    def body(dr_ref, dc_ref, out_ref):
        i = pl.program_id(0)

        @pl.when(i == 0)
        def _():
            out_ref[...] = jnp.zeros_like(out_ref)

        dr = dr_ref[...]
        dc = dc_ref[...]
        hi_oh = (jax.lax.broadcasted_iota(jnp.int32, (128, EB), 0)
                 == lax.shift_right_logical(dr, 7)).astype(jnp.bfloat16)
        lo_oh = (jax.lax.broadcasted_iota(jnp.int32, (EB, 128), 1)
                 == lax.bitwise_and(dc, 127)).astype(jnp.bfloat16)
        out_ref[...] += jnp.dot(hi_oh, lo_oh,
                                preferred_element_type=jnp.float32)

    return pl.pallas_call(
        body,
        grid=(E // EB,),
        in_specs=[
            pl.BlockSpec((1, EB), lambda i: (0, i)),
            pl.BlockSpec((EB, 1), lambda i: (i, 0)),
        ],
        out_specs=pl.BlockSpec((128, 128), lambda i: (0, 0)),
        out_shape=jax.ShapeDtypeStruct((128, 128), jnp.float32),
    )(dst_row, dst_col)


def _sc_aggregate(u, src3, dst3, zeros_pad):
    @functools.partial(
        pl.kernel,
        out_type=jax.ShapeDtypeStruct((NC, NPAD, 128), jnp.float32),
        mesh=_sc_mesh(),
        scratch_types=(
            [pltpu.VMEM((NCH // NST, C), jnp.int32),
             pltpu.VMEM((NCH // NST, C), jnp.int32)]
            + [pltpu.VMEM((C, 128), jnp.float32) for _ in range(NB)]
            + [pltpu.VMEM_SHARED((NPAD, 128), jnp.float32)]
            + [pltpu.SemaphoreType.DMA for _ in range(2 * NB)]
        ),
    )
    def agg_kernel(u_hbm, src_hbm, dst_hbm, z_hbm, out_hbm,
                   src_v, dst_v, *rest):
        rows = rest[:NB]
        acc = rest[NB]
        sem_g = rest[NB + 1:NB + 1 + NB]
        sem_s = rest[NB + 1 + NB:]
        cid = lax.axis_index("c")
        sid = lax.axis_index("s")
        wid = cid * NS + sid
        nps = NCH // NST
        pltpu.sync_copy(z_hbm.at[pl.ds(sid * RPT, RPT)],
                        acc.at[pl.ds(sid * RPT, RPT)])
        plsc.subcore_barrier()

        def wait_gather(b):
            pltpu.make_async_copy(u_hbm.at[pl.ds(0, C)], rows[b],
                                  sem_g[b]).wait()

        def wait_scatter(b):
            pltpu.make_async_copy(rows[b], acc.at[pl.ds(0, C)],
                                  sem_s[b]).wait()

        def stage(s, carry):
            pltpu.sync_copy(src_hbm.at[wid, pl.ds(s * nps, nps)], src_v)
            pltpu.sync_copy(dst_hbm.at[wid, pl.ds(s * nps, nps)], dst_v)
            for b in range(NB):
                pltpu.async_copy(u_hbm.at[src_v.at[b]], rows[b], sem_g[b])

            def grp(p, c2):
                j = NB * p
                for b in range(NB):
                    wait_gather(b)
                    pltpu.async_copy(rows[b], acc.at[dst_v.at[j + b]],
                                     sem_s[b], add=True)
                for b in range(NB):
                    wait_scatter(b)

                    @pl.when(p + 1 < nps // NB)
                    def _():
                        pltpu.async_copy(u_hbm.at[src_v.at[j + NB + b]],
                                         rows[b], sem_g[b])
                return c2

            lax.fori_loop(0, nps // NB, grp, 0)
            return carry

        lax.fori_loop(0, NST, stage, 0)
        plsc.subcore_barrier()
        pltpu.sync_copy(acc.at[pl.ds(sid * RPT, RPT)],
                        out_hbm.at[cid, pl.ds(sid * RPT, RPT)])

    return agg_kernel(u, src3, dst3, zeros_pad)


def _tc_layer1(degp, x, w1):
    def body(d_ref, x_ref, w_ref, u_ref, dinv_ref):
        deg = d_ref[...] + 1.0
        dinv = lax.rsqrt(jnp.maximum(deg, 1.0))
        xw = jnp.dot(x_ref[...], w_ref[...], preferred_element_type=jnp.float32)
        u_ref[...] = dinv * xw
        dinv_ref[...] = dinv

    return pl.pallas_call(
        body,
        grid=(N // BN,),
        in_specs=[
            pl.BlockSpec((BN, 1), lambda i: (i, 0)),
            pl.BlockSpec((BN, 128), lambda i: (i, 0)),
            pl.BlockSpec((128, 128), lambda i: (0, 0)),
        ],
        out_specs=[
            pl.BlockSpec((BN, 128), lambda i: (i, 0)),
            pl.BlockSpec((BN, 1), lambda i: (i, 0)),
        ],
        out_shape=[
            jax.ShapeDtypeStruct((N, 128), jnp.float32),
            jax.ShapeDtypeStruct((N, 1), jnp.float32),
        ],
    )(degp, x, w1)


def _tc_layer2(sp, u1, dinv, b1, w2):
    def body(s0_ref, s1_ref, u1_ref, dinv_ref, b_ref, w_ref, u2_ref):
        s = s0_ref[0] + s1_ref[0] + u1_ref[...]
        h = jnp.maximum(dinv_ref[...] * s + b_ref[...], 0.0)
        u2 = dinv_ref[...] * jnp.dot(
            h, w_ref[...], preferred_element_type=jnp.float32)
        u2_ref[...] = jnp.concatenate([u2, jnp.zeros_like(u2)], axis=1)

    return pl.pallas_call(
        body,
        grid=(N // BN,),
        in_specs=[
            pl.BlockSpec((1, BN, 128), lambda i: (0, i, 0)),
            pl.BlockSpec((1, BN, 128), lambda i: (1, i, 0)),
            pl.BlockSpec((BN, 128), lambda i: (i, 0)),
            pl.BlockSpec((BN, 1), lambda i: (i, 0)),
            pl.BlockSpec((1, 128), lambda i: (0, 0)),
            pl.BlockSpec((128, 64), lambda i: (0, 0)),
        ],
        out_specs=pl.BlockSpec((BN, 128), lambda i: (i, 0)),
        out_shape=jax.ShapeDtypeStruct((N, 128), jnp.float32),
    )(sp, sp, u1, dinv, b1.reshape(1, 128), w2)


def _tc_layer3(qp, u2, dinv, b2):
    def body(q0_ref, q1_ref, u2_ref, dinv_ref, b_ref, out_ref):
        t = (q0_ref[0] + q1_ref[0] + u2_ref[...])[:, :64]
        z = dinv_ref[...] * t + b_ref[...]
        m = jnp.max(z, axis=1, keepdims=True)
        lse = jnp.log(jnp.sum(jnp.exp(z - m), axis=1, keepdims=True)) + m
        out_ref[...] = z - lse

    return pl.pallas_call(
        body,
        grid=(N // BN,),
        in_specs=[
            pl.BlockSpec((1, BN, 128), lambda i: (0, i, 0)),
            pl.BlockSpec((1, BN, 128), lambda i: (1, i, 0)),
            pl.BlockSpec((BN, 128), lambda i: (i, 0)),
            pl.BlockSpec((BN, 1), lambda i: (i, 0)),
            pl.BlockSpec((1, 64), lambda i: (0, 0)),
        ],
        out_specs=pl.BlockSpec((BN, 64), lambda i: (i, 0)),
        out_shape=jax.ShapeDtypeStruct((N, 64), jnp.float32),
    )(qp, qp, u2, dinv, b2.reshape(1, 64))


def _pad_edges(edge_index):
    ppw = EPW - E // NW
    pad_src = jnp.zeros((NW, ppw), jnp.int32)
    pad_dst = jnp.broadcast_to(N + jnp.arange(ppw, dtype=jnp.int32), (NW, ppw))
    src3 = jnp.concatenate(
        [edge_index[0].reshape(NW, E // NW), pad_src], axis=1).reshape(NW, NCH, C)
    dst3 = jnp.concatenate(
        [edge_index[1].reshape(NW, E // NW), pad_dst], axis=1).reshape(NW, NCH, C)
    return src3, dst3


def kernel(x, edge_index, W1, b1, W2, b2):
    src3, dst3 = _pad_edges(edge_index)
    z128 = jnp.zeros((NPAD, 128), jnp.float32)

    dst = edge_index[1]
    flatdeg = _tc_histogram(dst.reshape(1, E), dst.reshape(E, 1))
    deg = flatdeg.reshape(128 * 128, 1)[:N]
    u1, dinv = _tc_layer1(deg, x, W1)
    sp = _sc_aggregate(u1, src3, dst3, z128)
    u2 = _tc_layer2(sp, u1, dinv, b1, W2)
    qp = _sc_aggregate(u2, src3, dst3, z128)
    return _tc_layer3(qp, u2, dinv, b2)


# speedup vs baseline: 1.3671x; 1.2067x over previous
import functools

import jax
import jax.numpy as jnp
from jax import lax
from jax.experimental import pallas as pl
from jax.experimental.pallas import tpu as pltpu
from jax.experimental.pallas import tpu_sc as plsc

N = 10000
E = 320000
NC, NS = 2, 16
NW = NC * NS
NPAD = 10240
C = 80
NCH = 128
NB = 4
NST = 4
EPW = NCH * C
EPAD = NW * EPW
NROW = NPAD // C
RPT = NPAD // NS
BN = 400


def _sc_mesh():
    return plsc.VectorSubcoreMesh(core_axis_name="c", subcore_axis_name="s")


EB = 32000


def _tc_histogram(dst_row, dst_col):
    def body(dr_ref, dc_ref, out_ref):
        i = pl.program_id(0)

        @pl.when(i == 0)
        def _():
            out_ref[...] = jnp.zeros_like(out_ref)

        dr = dr_ref[...]
        dc = dc_ref[...]
        hi_oh = (jax.lax.broadcasted_iota(jnp.int32, (128, EB), 0)
                 == lax.shift_right_logical(dr, 7)).astype(jnp.bfloat16)
        lo_oh = (jax.lax.broadcasted_iota(jnp.int32, (EB, 128), 1)
                 == lax.bitwise_and(dc, 127)).astype(jnp.bfloat16)
        out_ref[...] += jnp.dot(hi_oh, lo_oh,
                                preferred_element_type=jnp.float32)

    return pl.pallas_call(
        body,
        grid=(E // EB,),
        in_specs=[
            pl.BlockSpec((1, EB), lambda i: (0, i)),
            pl.BlockSpec((EB, 1), lambda i: (i, 0)),
        ],
        out_specs=pl.BlockSpec((128, 128), lambda i: (0, 0)),
        out_shape=jax.ShapeDtypeStruct((128, 128), jnp.float32),
    )(dst_row, dst_col)


def _sc_aggregate(u, src3, dst3, zeros_pad, d=128):
    @functools.partial(
        pl.kernel,
        out_type=jax.ShapeDtypeStruct((NC, NPAD, d), jnp.float32),
        mesh=_sc_mesh(),
        compiler_params=(None if d == 128 else
                         pltpu.CompilerParams(use_tc_tiling_on_sc=False)),
        scratch_types=(
            [pltpu.VMEM((NCH // NST, C), jnp.int32),
             pltpu.VMEM((NCH // NST, C), jnp.int32)]
            + [pltpu.VMEM((C, d), jnp.float32) for _ in range(NB)]
            + [pltpu.VMEM_SHARED((NPAD, d), jnp.float32)]
            + [pltpu.SemaphoreType.DMA for _ in range(2 * NB)]
        ),
    )
    def agg_kernel(u_hbm, src_hbm, dst_hbm, z_hbm, out_hbm,
                   src_v, dst_v, *rest):
        rows = rest[:NB]
        acc = rest[NB]
        sem_g = rest[NB + 1:NB + 1 + NB]
        sem_s = rest[NB + 1 + NB:]
        cid = lax.axis_index("c")
        sid = lax.axis_index("s")
        wid = cid * NS + sid
        nps = NCH // NST
        pltpu.sync_copy(z_hbm.at[pl.ds(sid * RPT, RPT)],
                        acc.at[pl.ds(sid * RPT, RPT)])
        plsc.subcore_barrier()

        def wait_gather(b):
            pltpu.make_async_copy(u_hbm.at[pl.ds(0, C)], rows[b],
                                  sem_g[b]).wait()

        def wait_scatter(b):
            pltpu.make_async_copy(rows[b], acc.at[pl.ds(0, C)],
                                  sem_s[b]).wait()

        def stage(s, carry):
            pltpu.sync_copy(src_hbm.at[wid, pl.ds(s * nps, nps)], src_v)
            pltpu.sync_copy(dst_hbm.at[wid, pl.ds(s * nps, nps)], dst_v)
            for b in range(NB):
                pltpu.async_copy(u_hbm.at[src_v.at[b]], rows[b], sem_g[b])

            def grp(p, c2):
                j = NB * p
                for b in range(NB):
                    wait_gather(b)
                    pltpu.async_copy(rows[b], acc.at[dst_v.at[j + b]],
                                     sem_s[b], add=True)
                for b in range(NB):
                    wait_scatter(b)

                    @pl.when(p + 1 < nps // NB)
                    def _():
                        pltpu.async_copy(u_hbm.at[src_v.at[j + NB + b]],
                                         rows[b], sem_g[b])
                return c2

            lax.fori_loop(0, nps // NB, grp, 0)
            return carry

        lax.fori_loop(0, NST, stage, 0)
        plsc.subcore_barrier()
        pltpu.sync_copy(acc.at[pl.ds(sid * RPT, RPT)],
                        out_hbm.at[cid, pl.ds(sid * RPT, RPT)])

    return agg_kernel(u, src3, dst3, zeros_pad)


def _tc_layer1(degp, x, w1):
    def body(d_ref, x_ref, w_ref, u_ref, dinv_ref):
        deg = d_ref[...] + 1.0
        dinv = lax.rsqrt(jnp.maximum(deg, 1.0))
        xw = jnp.dot(x_ref[...], w_ref[...], preferred_element_type=jnp.float32)
        u_ref[...] = dinv * xw
        dinv_ref[...] = dinv

    return pl.pallas_call(
        body,
        grid=(N // BN,),
        in_specs=[
            pl.BlockSpec((BN, 1), lambda i: (i, 0)),
            pl.BlockSpec((BN, 128), lambda i: (i, 0)),
            pl.BlockSpec((128, 128), lambda i: (0, 0)),
        ],
        out_specs=[
            pl.BlockSpec((BN, 128), lambda i: (i, 0)),
            pl.BlockSpec((BN, 1), lambda i: (i, 0)),
        ],
        out_shape=[
            jax.ShapeDtypeStruct((N, 128), jnp.float32),
            jax.ShapeDtypeStruct((N, 1), jnp.float32),
        ],
    )(degp, x, w1)


def _tc_layer2(sp, u1, dinv, b1, w2):
    def body(s0_ref, s1_ref, u1_ref, dinv_ref, b_ref, w_ref, u2_ref):
        s = s0_ref[0] + s1_ref[0] + u1_ref[...]
        h = jnp.maximum(dinv_ref[...] * s + b_ref[...], 0.0)
        u2_ref[...] = dinv_ref[...] * jnp.dot(
            h, w_ref[...], preferred_element_type=jnp.float32)

    return pl.pallas_call(
        body,
        grid=(N // BN,),
        in_specs=[
            pl.BlockSpec((1, BN, 128), lambda i: (0, i, 0)),
            pl.BlockSpec((1, BN, 128), lambda i: (1, i, 0)),
            pl.BlockSpec((BN, 128), lambda i: (i, 0)),
            pl.BlockSpec((BN, 1), lambda i: (i, 0)),
            pl.BlockSpec((1, 128), lambda i: (0, 0)),
            pl.BlockSpec((128, 64), lambda i: (0, 0)),
        ],
        out_specs=pl.BlockSpec((BN, 64), lambda i: (i, 0)),
        out_shape=jax.ShapeDtypeStruct((N, 64), jnp.float32),
    )(sp, sp, u1, dinv, b1.reshape(1, 128), w2)


def _tc_layer3(qp, u2, dinv, b2):
    def body(q0_ref, q1_ref, u2_ref, dinv_ref, b_ref, out_ref):
        z = (dinv_ref[...] * (q0_ref[0] + q1_ref[0] + u2_ref[...])
             + b_ref[...])
        m = jnp.max(z, axis=1, keepdims=True)
        lse = jnp.log(jnp.sum(jnp.exp(z - m), axis=1, keepdims=True)) + m
        out_ref[...] = z - lse

    return pl.pallas_call(
        body,
        grid=(N // BN,),
        in_specs=[
            pl.BlockSpec((1, BN, 64), lambda i: (0, i, 0)),
            pl.BlockSpec((1, BN, 64), lambda i: (1, i, 0)),
            pl.BlockSpec((BN, 64), lambda i: (i, 0)),
            pl.BlockSpec((BN, 1), lambda i: (i, 0)),
            pl.BlockSpec((1, 64), lambda i: (0, 0)),
        ],
        out_specs=pl.BlockSpec((BN, 64), lambda i: (i, 0)),
        out_shape=jax.ShapeDtypeStruct((N, 64), jnp.float32),
    )(qp, qp, u2, dinv, b2.reshape(1, 64))


def _pad_edges(edge_index):
    ppw = EPW - E // NW
    pad_src = jnp.zeros((NW, ppw), jnp.int32)
    pad_dst = jnp.broadcast_to(N + jnp.arange(ppw, dtype=jnp.int32), (NW, ppw))
    src3 = jnp.concatenate(
        [edge_index[0].reshape(NW, E // NW), pad_src], axis=1).reshape(NW, NCH, C)
    dst3 = jnp.concatenate(
        [edge_index[1].reshape(NW, E // NW), pad_dst], axis=1).reshape(NW, NCH, C)
    return src3, dst3


def kernel(x, edge_index, W1, b1, W2, b2):
    src3, dst3 = _pad_edges(edge_index)
    z128 = jnp.zeros((NPAD, 128), jnp.float32)
    z64 = jnp.zeros((NPAD, 64), jnp.float32)

    dst = edge_index[1]
    flatdeg = _tc_histogram(dst.reshape(1, E), dst.reshape(E, 1))
    deg = flatdeg.reshape(128 * 128, 1)[:N]
    u1, dinv = _tc_layer1(deg, x, W1)
    sp = _sc_aggregate(u1, src3, dst3, z128)
    u2 = _tc_layer2(sp, u1, dinv, b1, W2)
    qp = _sc_aggregate(u2, src3, dst3, z64, 64)
    return _tc_layer3(qp, u2, dinv, b2)


# speedup vs baseline: 1.6066x; 1.1752x over previous
import functools

import jax
import jax.numpy as jnp
from jax import lax
from jax.experimental import pallas as pl
from jax.experimental.pallas import tpu as pltpu
from jax.experimental.pallas import tpu_sc as plsc

N = 10000
E = 320000
NC, NS = 2, 16
NW = NC * NS
NPAD = 10240
C = 80
NCH = 128
NB = 4
NST = 4
EPW = NCH * C
EPAD = NW * EPW
NROW = NPAD // C
RPT = NPAD // NS
BN = 1000


def _sc_mesh():
    return plsc.VectorSubcoreMesh(core_axis_name="c", subcore_axis_name="s")


EB = 32000


def _tc_histogram(dst_row):
    def body(dr_ref, out_ref):
        i = pl.program_id(0)

        @pl.when(i == 0)
        def _():
            out_ref[...] = jnp.zeros_like(out_ref)

        dr = dr_ref[...]
        subl = jax.lax.broadcasted_iota(jnp.int32, (128, EB), 0)
        hi_oh = (subl == lax.shift_right_logical(dr, 7)).astype(jnp.bfloat16)
        lo_oh = (subl == lax.bitwise_and(dr, 127)).astype(jnp.bfloat16)
        out_ref[...] += lax.dot_general(
            hi_oh, lo_oh, (((1,), (1,)), ((), ())),
            preferred_element_type=jnp.float32)

    return pl.pallas_call(
        body,
        grid=(E // EB,),
        in_specs=[pl.BlockSpec((1, EB), lambda i: (0, i))],
        out_specs=pl.BlockSpec((128, 128), lambda i: (0, 0)),
        out_shape=jax.ShapeDtypeStruct((128, 128), jnp.float32),
    )(dst_row)


def _sc_aggregate(u, src3, dst3, zeros_pad, d=128):
    @functools.partial(
        pl.kernel,
        out_type=jax.ShapeDtypeStruct((NC, NPAD, d), jnp.float32),
        mesh=_sc_mesh(),
        compiler_params=(None if d == 128 else
                         pltpu.CompilerParams(use_tc_tiling_on_sc=False)),
        scratch_types=(
            [pltpu.VMEM((NCH // NST, C), jnp.int32),
             pltpu.VMEM((NCH // NST, C), jnp.int32)]
            + [pltpu.VMEM((C, d), jnp.float32) for _ in range(NB)]
            + [pltpu.VMEM_SHARED((NPAD, d), jnp.float32)]
            + [pltpu.SemaphoreType.DMA for _ in range(2 * NB)]
        ),
    )
    def agg_kernel(u_hbm, src_hbm, dst_hbm, z_hbm, out_hbm,
                   src_v, dst_v, *rest):
        rows = rest[:NB]
        acc = rest[NB]
        sem_g = rest[NB + 1:NB + 1 + NB]
        sem_s = rest[NB + 1 + NB:]
        cid = lax.axis_index("c")
        sid = lax.axis_index("s")
        wid = cid * NS + sid
        nps = NCH // NST
        pltpu.sync_copy(z_hbm.at[pl.ds(sid * RPT, RPT)],
                        acc.at[pl.ds(sid * RPT, RPT)])
        plsc.subcore_barrier()

        def wait_gather(b):
            pltpu.make_async_copy(u_hbm.at[pl.ds(0, C)], rows[b],
                                  sem_g[b]).wait()

        def wait_scatter(b):
            pltpu.make_async_copy(rows[b], acc.at[pl.ds(0, C)],
                                  sem_s[b]).wait()

        def stage(s, carry):
            pltpu.sync_copy(src_hbm.at[wid, pl.ds(s * nps, nps)], src_v)
            pltpu.sync_copy(dst_hbm.at[wid, pl.ds(s * nps, nps)], dst_v)
            for b in range(NB):
                pltpu.async_copy(u_hbm.at[src_v.at[b]], rows[b], sem_g[b])

            def grp(p, c2):
                j = NB * p
                for b in range(NB):
                    wait_gather(b)
                    pltpu.async_copy(rows[b], acc.at[dst_v.at[j + b]],
                                     sem_s[b], add=True)
                for b in range(NB):
                    wait_scatter(b)

                    @pl.when(p + 1 < nps // NB)
                    def _():
                        pltpu.async_copy(u_hbm.at[src_v.at[j + NB + b]],
                                         rows[b], sem_g[b])
                return c2

            lax.fori_loop(0, nps // NB, grp, 0)
            return carry

        lax.fori_loop(0, NST, stage, 0)
        plsc.subcore_barrier()
        pltpu.sync_copy(acc.at[pl.ds(sid * RPT, RPT)],
                        out_hbm.at[cid, pl.ds(sid * RPT, RPT)])

    return agg_kernel(u, src3, dst3, zeros_pad)


def _tc_layer1(flatdeg, x, w1):
    def body(fd_ref, x_ref, w_ref, u_ref, dinv_ref):
        i = pl.program_id(0)
        n = i * BN + jax.lax.broadcasted_iota(jnp.int32, (BN, 128), 0)
        lanes = jax.lax.broadcasted_iota(jnp.int32, (BN, 128), 1)
        sel = (lanes == lax.shift_right_logical(n, 7)).astype(jnp.float32)
        mix = jnp.dot(sel, fd_ref[...], preferred_element_type=jnp.float32)
        mask = (lanes == lax.bitwise_and(n, 127)).astype(jnp.float32)
        deg = jnp.sum(mix * mask, axis=1, keepdims=True)
        dinv = lax.rsqrt(deg + 1.0)
        xw = jnp.dot(x_ref[...], w_ref[...], preferred_element_type=jnp.float32)
        u_ref[...] = dinv * xw
        dinv_ref[...] = jnp.broadcast_to(dinv, (BN, 128))

    return pl.pallas_call(
        body,
        grid=(N // BN,),
        in_specs=[
            pl.BlockSpec((128, 128), lambda i: (0, 0)),
            pl.BlockSpec((BN, 128), lambda i: (i, 0)),
            pl.BlockSpec((128, 128), lambda i: (0, 0)),
        ],
        out_specs=[
            pl.BlockSpec((BN, 128), lambda i: (i, 0)),
            pl.BlockSpec((BN, 128), lambda i: (i, 0)),
        ],
        out_shape=[
            jax.ShapeDtypeStruct((N, 128), jnp.float32),
            jax.ShapeDtypeStruct((N, 128), jnp.float32),
        ],
    )(flatdeg, x, w1)


def _tc_layer2(sp, u1, dinv, b1, w2):
    def body(s0_ref, s1_ref, u1_ref, dinv_ref, b_ref, w_ref, u2_ref):
        s = s0_ref[0] + s1_ref[0] + u1_ref[...]
        h = jnp.maximum(dinv_ref[...] * s + b_ref[...], 0.0)
        u2_ref[...] = dinv_ref[:, :64] * jnp.dot(
            h, w_ref[...], preferred_element_type=jnp.float32)

    return pl.pallas_call(
        body,
        grid=(N // BN,),
        in_specs=[
            pl.BlockSpec((1, BN, 128), lambda i: (0, i, 0)),
            pl.BlockSpec((1, BN, 128), lambda i: (1, i, 0)),
            pl.BlockSpec((BN, 128), lambda i: (i, 0)),
            pl.BlockSpec((BN, 128), lambda i: (i, 0)),
            pl.BlockSpec((1, 128), lambda i: (0, 0)),
            pl.BlockSpec((128, 64), lambda i: (0, 0)),
        ],
        out_specs=pl.BlockSpec((BN, 64), lambda i: (i, 0)),
        out_shape=jax.ShapeDtypeStruct((N, 64), jnp.float32),
    )(sp, sp, u1, dinv, b1.reshape(1, 128), w2)


def _tc_layer3(qp, u2, dinv, b2):
    def body(q0_ref, q1_ref, u2_ref, dinv_ref, b_ref, out_ref):
        z = (dinv_ref[:, :64] * (q0_ref[0] + q1_ref[0] + u2_ref[...])
             + b_ref[...])
        m = jnp.max(z, axis=1, keepdims=True)
        lse = jnp.log(jnp.sum(jnp.exp(z - m), axis=1, keepdims=True)) + m
        out_ref[...] = z - lse

    return pl.pallas_call(
        body,
        grid=(N // BN,),
        in_specs=[
            pl.BlockSpec((1, BN, 64), lambda i: (0, i, 0)),
            pl.BlockSpec((1, BN, 64), lambda i: (1, i, 0)),
            pl.BlockSpec((BN, 64), lambda i: (i, 0)),
            pl.BlockSpec((BN, 128), lambda i: (i, 0)),
            pl.BlockSpec((1, 64), lambda i: (0, 0)),
        ],
        out_specs=pl.BlockSpec((BN, 64), lambda i: (i, 0)),
        out_shape=jax.ShapeDtypeStruct((N, 64), jnp.float32),
    )(qp, qp, u2, dinv, b2.reshape(1, 64))


def _pad_edges(edge_index):
    ppw = EPW - E // NW
    pad_src = jnp.zeros((NW, ppw), jnp.int32)
    pad_dst = jnp.broadcast_to(N + jnp.arange(ppw, dtype=jnp.int32), (NW, ppw))
    src3 = jnp.concatenate(
        [edge_index[0].reshape(NW, E // NW), pad_src], axis=1).reshape(NW, NCH, C)
    dst3 = jnp.concatenate(
        [edge_index[1].reshape(NW, E // NW), pad_dst], axis=1).reshape(NW, NCH, C)
    return src3, dst3


def kernel(x, edge_index, W1, b1, W2, b2):
    src3, dst3 = _pad_edges(edge_index)
    z128 = jnp.zeros((NPAD, 128), jnp.float32)
    z64 = jnp.zeros((NPAD, 64), jnp.float32)

    flatdeg = _tc_histogram(edge_index[1].reshape(1, E))
    u1, dinv = _tc_layer1(flatdeg, x, W1)
    sp = _sc_aggregate(u1, src3, dst3, z128)
    u2 = _tc_layer2(sp, u1, dinv, b1, W2)
    qp = _sc_aggregate(u2, src3, dst3, z64, 64)
    return _tc_layer3(qp, u2, dinv, b2)


# speedup vs baseline: 1.6158x; 1.0057x over previous
import functools

import jax
import jax.numpy as jnp
from jax import lax
from jax.experimental import pallas as pl
from jax.experimental.pallas import tpu as pltpu
from jax.experimental.pallas import tpu_sc as plsc

N = 10000
E = 320000
NC, NS = 2, 16
NW = NC * NS
NPAD = 10240
C = 80
NCH = 128
NB = 4
NST = 4
EPW = NCH * C
EPAD = NW * EPW
NROW = NPAD // C
RPT = NPAD // NS
BN = 1000


def _sc_mesh():
    return plsc.VectorSubcoreMesh(core_axis_name="c", subcore_axis_name="s")


EB = 32000


def _tc_histogram(edge_index):
    def body(ei_ref, out_ref):
        i = pl.program_id(0)

        @pl.when(i == 0)
        def _():
            out_ref[...] = jnp.zeros_like(out_ref)

        dr = ei_ref[1:2, :]
        subl = jax.lax.broadcasted_iota(jnp.int32, (128, EB), 0)
        hi_oh = (subl == lax.shift_right_logical(dr, 7)).astype(jnp.bfloat16)
        lo_oh = (subl == lax.bitwise_and(dr, 127)).astype(jnp.bfloat16)
        out_ref[...] += lax.dot_general(
            hi_oh, lo_oh, (((1,), (1,)), ((), ())),
            preferred_element_type=jnp.float32)

    return pl.pallas_call(
        body,
        grid=(E // EB,),
        in_specs=[pl.BlockSpec((2, EB), lambda i: (0, i))],
        out_specs=pl.BlockSpec((128, 128), lambda i: (0, 0)),
        out_shape=jax.ShapeDtypeStruct((128, 128), jnp.float32),
    )(edge_index)


def _sc_aggregate(u, src3, dst3, zeros_pad, d=128):
    @functools.partial(
        pl.kernel,
        out_type=jax.ShapeDtypeStruct((NC, NPAD, d), jnp.float32),
        mesh=_sc_mesh(),
        compiler_params=(None if d == 128 else
                         pltpu.CompilerParams(use_tc_tiling_on_sc=False)),
        scratch_types=(
            [pltpu.VMEM((NCH // NST, C), jnp.int32),
             pltpu.VMEM((NCH // NST, C), jnp.int32)]
            + [pltpu.VMEM((C, d), jnp.float32) for _ in range(NB)]
            + [pltpu.VMEM_SHARED((NPAD, d), jnp.float32)]
            + [pltpu.SemaphoreType.DMA for _ in range(2 * NB)]
        ),
    )
    def agg_kernel(u_hbm, src_hbm, dst_hbm, z_hbm, out_hbm,
                   src_v, dst_v, *rest):
        rows = rest[:NB]
        acc = rest[NB]
        sem_g = rest[NB + 1:NB + 1 + NB]
        sem_s = rest[NB + 1 + NB:]
        cid = lax.axis_index("c")
        sid = lax.axis_index("s")
        wid = cid * NS + sid
        nps = NCH // NST
        pltpu.sync_copy(z_hbm.at[pl.ds(sid * RPT, RPT)],
                        acc.at[pl.ds(sid * RPT, RPT)])
        plsc.subcore_barrier()

        def wait_gather(b):
            pltpu.make_async_copy(u_hbm.at[pl.ds(0, C)], rows[b],
                                  sem_g[b]).wait()

        def wait_scatter(b):
            pltpu.make_async_copy(rows[b], acc.at[pl.ds(0, C)],
                                  sem_s[b]).wait()

        def stage(s, carry):
            pltpu.sync_copy(src_hbm.at[wid, pl.ds(s * nps, nps)], src_v)
            pltpu.sync_copy(dst_hbm.at[wid, pl.ds(s * nps, nps)], dst_v)
            for b in range(NB):
                pltpu.async_copy(u_hbm.at[src_v.at[b]], rows[b], sem_g[b])

            def grp(p, c2):
                j = NB * p
                for b in range(NB):
                    wait_gather(b)
                    pltpu.async_copy(rows[b], acc.at[dst_v.at[j + b]],
                                     sem_s[b], add=True)
                for b in range(NB):
                    wait_scatter(b)

                    @pl.when(p + 1 < nps // NB)
                    def _():
                        pltpu.async_copy(u_hbm.at[src_v.at[j + NB + b]],
                                         rows[b], sem_g[b])
                return c2

            lax.fori_loop(0, nps // NB, grp, 0)
            return carry

        lax.fori_loop(0, NST, stage, 0)
        plsc.subcore_barrier()
        pltpu.sync_copy(acc.at[pl.ds(sid * RPT, RPT)],
                        out_hbm.at[cid, pl.ds(sid * RPT, RPT)])

    return agg_kernel(u, src3, dst3, zeros_pad)


def _tc_layer1(flatdeg, x, w1):
    def body(fd_ref, x_ref, w_ref, u_ref, dinv_ref):
        i = pl.program_id(0)
        n = i * BN + jax.lax.broadcasted_iota(jnp.int32, (BN, 128), 0)
        lanes = jax.lax.broadcasted_iota(jnp.int32, (BN, 128), 1)
        sel = (lanes == lax.shift_right_logical(n, 7)).astype(jnp.float32)
        mix = jnp.dot(sel, fd_ref[...], preferred_element_type=jnp.float32)
        mask = (lanes == lax.bitwise_and(n, 127)).astype(jnp.float32)
        deg = jnp.sum(mix * mask, axis=1, keepdims=True)
        dinv = lax.rsqrt(deg + 1.0)
        xw = jnp.dot(x_ref[...], w_ref[...], preferred_element_type=jnp.float32)
        u_ref[...] = dinv * xw
        dinv_ref[...] = jnp.broadcast_to(dinv, (BN, 128))

    return pl.pallas_call(
        body,
        grid=(N // BN,),
        in_specs=[
            pl.BlockSpec((128, 128), lambda i: (0, 0)),
            pl.BlockSpec((BN, 128), lambda i: (i, 0)),
            pl.BlockSpec((128, 128), lambda i: (0, 0)),
        ],
        out_specs=[
            pl.BlockSpec((BN, 128), lambda i: (i, 0)),
            pl.BlockSpec((BN, 128), lambda i: (i, 0)),
        ],
        out_shape=[
            jax.ShapeDtypeStruct((N, 128), jnp.float32),
            jax.ShapeDtypeStruct((N, 128), jnp.float32),
        ],
    )(flatdeg, x, w1)


def _tc_layer2(sp, u1, dinv, b1, w2):
    def body(s0_ref, s1_ref, u1_ref, dinv_ref, b_ref, w_ref, u2_ref):
        s = s0_ref[0] + s1_ref[0] + u1_ref[...]
        h = jnp.maximum(dinv_ref[...] * s + b_ref[...], 0.0)
        u2_ref[...] = dinv_ref[:, :64] * jnp.dot(
            h, w_ref[...], preferred_element_type=jnp.float32)

    return pl.pallas_call(
        body,
        grid=(N // BN,),
        in_specs=[
            pl.BlockSpec((1, BN, 128), lambda i: (0, i, 0)),
            pl.BlockSpec((1, BN, 128), lambda i: (1, i, 0)),
            pl.BlockSpec((BN, 128), lambda i: (i, 0)),
            pl.BlockSpec((BN, 128), lambda i: (i, 0)),
            pl.BlockSpec((1, 128), lambda i: (0, 0)),
            pl.BlockSpec((128, 64), lambda i: (0, 0)),
        ],
        out_specs=pl.BlockSpec((BN, 64), lambda i: (i, 0)),
        out_shape=jax.ShapeDtypeStruct((N, 64), jnp.float32),
    )(sp, sp, u1, dinv, b1.reshape(1, 128), w2)


def _tc_layer3(qp, u2, dinv, b2):
    def body(q0_ref, q1_ref, u2_ref, dinv_ref, b_ref, out_ref):
        z = (dinv_ref[:, :64] * (q0_ref[0] + q1_ref[0] + u2_ref[...])
             + b_ref[...])
        m = jnp.max(z, axis=1, keepdims=True)
        lse = jnp.log(jnp.sum(jnp.exp(z - m), axis=1, keepdims=True)) + m
        out_ref[...] = z - lse

    return pl.pallas_call(
        body,
        grid=(N // BN,),
        in_specs=[
            pl.BlockSpec((1, BN, 64), lambda i: (0, i, 0)),
            pl.BlockSpec((1, BN, 64), lambda i: (1, i, 0)),
            pl.BlockSpec((BN, 64), lambda i: (i, 0)),
            pl.BlockSpec((BN, 128), lambda i: (i, 0)),
            pl.BlockSpec((1, 64), lambda i: (0, 0)),
        ],
        out_specs=pl.BlockSpec((BN, 64), lambda i: (i, 0)),
        out_shape=jax.ShapeDtypeStruct((N, 64), jnp.float32),
    )(qp, qp, u2, dinv, b2.reshape(1, 64))


def _pad_edges(edge_index):
    ppw = EPW - E // NW
    pad_src = jnp.zeros((NW, ppw), jnp.int32)
    pad_dst = jnp.broadcast_to(N + jnp.arange(ppw, dtype=jnp.int32), (NW, ppw))
    src3 = jnp.concatenate(
        [edge_index[0].reshape(NW, E // NW), pad_src], axis=1).reshape(NW, NCH, C)
    dst3 = jnp.concatenate(
        [edge_index[1].reshape(NW, E // NW), pad_dst], axis=1).reshape(NW, NCH, C)
    return src3, dst3


def kernel(x, edge_index, W1, b1, W2, b2):
    src3, dst3 = _pad_edges(edge_index)
    z128 = jnp.zeros((NPAD, 128), jnp.float32)
    z64 = jnp.zeros((NPAD, 64), jnp.float32)

    flatdeg = _tc_histogram(edge_index)
    u1, dinv = _tc_layer1(flatdeg, x, W1)
    sp = _sc_aggregate(u1, src3, dst3, z128)
    u2 = _tc_layer2(sp, u1, dinv, b1, W2)
    qp = _sc_aggregate(u2, src3, dst3, z64, 64)
    return _tc_layer3(qp, u2, dinv, b2)
